# baseline (speedup 1.0000x reference)
import jax
import jax.numpy as jnp
from jax.experimental import pallas as pl
from jax.experimental.pallas import tpu as pltpu

_TM = 512


def _chamfer_body(x1_ref, x2t_ref, dist1_ref, idx1_ref, dist2_ref, idx2_ref):
    mt = pl.program_id(1)
    x1 = x1_ref[0]
    x2t = x2t_ref[0]
    dx = x1[:, 0:1] - x2t[0:1, :]
    dy = x1[:, 1:2] - x2t[1:2, :]
    dz = x1[:, 2:3] - x2t[2:3, :]
    d = (dx * dx + dy * dy) + dz * dz

    dist2_ref[0, 0, :] = jnp.min(d, axis=0)
    idx2_ref[0, 0, :] = jnp.argmin(d, axis=0).astype(jnp.int32)

    lmin = jnp.min(d, axis=1)
    larg = jnp.argmin(d, axis=1).astype(jnp.int32) + mt * _TM

    @pl.when(mt == 0)
    def _():
        dist1_ref[0, 0, :] = lmin
        idx1_ref[0, 0, :] = larg

    @pl.when(mt != 0)
    def _():
        prev = dist1_ref[0, 0, :]
        previ = idx1_ref[0, 0, :]
        better = lmin < prev
        dist1_ref[0, 0, :] = jnp.where(better, lmin, prev)
        idx1_ref[0, 0, :] = jnp.where(better, larg, previ)


def kernel(xyz1, xyz2):
    B, N, _ = xyz1.shape
    M = xyz2.shape[1]
    x2t = jnp.transpose(xyz2, (0, 2, 1))
    grid = (B, M // _TM)

    dist1, idx1, dist2, idx2 = pl.pallas_call(
        _chamfer_body,
        grid=grid,
        in_specs=[
            pl.BlockSpec((1, N, 3), lambda b, mt: (b, 0, 0)),
            pl.BlockSpec((1, 3, _TM), lambda b, mt: (b, 0, mt)),
        ],
        out_specs=[
            pl.BlockSpec((1, 1, N), lambda b, mt: (b, 0, 0)),
            pl.BlockSpec((1, 1, N), lambda b, mt: (b, 0, 0)),
            pl.BlockSpec((1, 1, _TM), lambda b, mt: (b, 0, mt)),
            pl.BlockSpec((1, 1, _TM), lambda b, mt: (b, 0, mt)),
        ],
        out_shape=[
            jax.ShapeDtypeStruct((B, 1, N), jnp.float32),
            jax.ShapeDtypeStruct((B, 1, N), jnp.int32),
            jax.ShapeDtypeStruct((B, 1, M), jnp.float32),
            jax.ShapeDtypeStruct((B, 1, M), jnp.int32),
        ],
        compiler_params=pltpu.CompilerParams(
            dimension_semantics=("parallel", "arbitrary"),
        ),
    )(xyz1, x2t)

    return (
        dist1.reshape(B, N),
        dist2.reshape(B, M),
        idx1.reshape(B, N),
        idx2.reshape(B, M),
    )


# speedup vs baseline: 2.1912x; 2.1912x over previous
import jax
import jax.numpy as jnp
from jax import lax
from jax.experimental import pallas as pl
from jax.experimental.pallas import tpu as pltpu

_TM = 512
_BIG = 2**30


def _chamfer_body(x1_ref, x2t_ref, dist1_ref, idx1_ref, dist2_ref, idx2_ref,
                  qv_s, qi_s):
    mt = pl.program_id(1)
    nmt = pl.num_programs(1)
    n = x1_ref.shape[1]
    x1 = x1_ref[0]
    x2t = x2t_ref[0]
    dx = x1[:, 0:1] - x2t[0:1, :]
    dy = x1[:, 1:2] - x2t[1:2, :]
    dz = x1[:, 2:3] - x2t[2:3, :]
    d = (dx * dx + dy * dy) + dz * dz

    d3 = d.reshape(n // 8, 8, _TM)
    pv = jnp.min(d3, axis=0)
    pa = jnp.argmin(d3, axis=0).astype(jnp.int32)
    rowidx = pa * 8 + lax.broadcasted_iota(jnp.int32, (8, _TM), 0)
    fv = jnp.min(pv, axis=0)
    fi = jnp.min(jnp.where(pv == fv[None, :], rowidx, _BIG), axis=0)
    dist2_ref[0, 0, :] = fv
    idx2_ref[0, 0, :] = fi

    lane = lax.broadcasted_iota(jnp.int32, (n, 128), 1)
    qv = d[:, 0:128]
    qi = lane + mt * _TM
    for t in range(1, _TM // 128):
        dt = d[:, t * 128:(t + 1) * 128]
        m = dt < qv
        qv = jnp.where(m, dt, qv)
        qi = jnp.where(m, lane + (t * 128 + mt * _TM), qi)

    @pl.when(mt == 0)
    def _():
        qv_s[...] = qv
        qi_s[...] = qi

    @pl.when(mt != 0)
    def _():
        pv1 = qv_s[...]
        m2 = qv < pv1
        qv_s[...] = jnp.where(m2, qv, pv1)
        qi_s[...] = jnp.where(m2, qi, qi_s[...])

    @pl.when(mt == nmt - 1)
    def _():
        qvf = qv_s[...]
        qif = qi_s[...]
        gv = jnp.min(qvf, axis=1)
        gi = jnp.min(jnp.where(qvf == gv[:, None], qif, _BIG), axis=1)
        dist1_ref[0, 0, :] = gv
        idx1_ref[0, 0, :] = gi


def kernel(xyz1, xyz2):
    B, N, _ = xyz1.shape
    M = xyz2.shape[1]
    x2t = jnp.transpose(xyz2, (0, 2, 1))
    grid = (B, M // _TM)

    dist1, idx1, dist2, idx2 = pl.pallas_call(
        _chamfer_body,
        grid=grid,
        in_specs=[
            pl.BlockSpec((1, N, 3), lambda b, mt: (b, 0, 0)),
            pl.BlockSpec((1, 3, _TM), lambda b, mt: (b, 0, mt)),
        ],
        out_specs=[
            pl.BlockSpec((1, 1, N), lambda b, mt: (b, 0, 0)),
            pl.BlockSpec((1, 1, N), lambda b, mt: (b, 0, 0)),
            pl.BlockSpec((1, 1, _TM), lambda b, mt: (b, 0, mt)),
            pl.BlockSpec((1, 1, _TM), lambda b, mt: (b, 0, mt)),
        ],
        out_shape=[
            jax.ShapeDtypeStruct((B, 1, N), jnp.float32),
            jax.ShapeDtypeStruct((B, 1, N), jnp.int32),
            jax.ShapeDtypeStruct((B, 1, M), jnp.float32),
            jax.ShapeDtypeStruct((B, 1, M), jnp.int32),
        ],
        scratch_shapes=[
            pltpu.VMEM((N, 128), jnp.float32),
            pltpu.VMEM((N, 128), jnp.int32),
        ],
        compiler_params=pltpu.CompilerParams(
            dimension_semantics=("parallel", "arbitrary"),
        ),
    )(xyz1, x2t)

    return (
        dist1.reshape(B, N),
        dist2.reshape(B, M),
        idx1.reshape(B, N),
        idx2.reshape(B, M),
    )


# speedup vs baseline: 2.4939x; 1.1382x over previous
import jax
import jax.numpy as jnp
from jax import lax
from jax.experimental import pallas as pl
from jax.experimental.pallas import tpu as pltpu

_TM = 512
_RC = 64
_BIG = 2**30


def _chamfer_body(x1_ref, x2t_ref, dist1_ref, idx1_ref, dist2_ref, idx2_ref,
                  qv_s, qi_s):
    mt = pl.program_id(1)
    nmt = pl.num_programs(1)
    n = x1_ref.shape[1]
    x2t = x2t_ref[0]
    x2x = x2t[0:1, :]
    x2y = x2t[1:2, :]
    x2z = x2t[2:3, :]
    lane = lax.broadcasted_iota(jnp.int32, (_RC, 128), 1)

    @pl.when(mt == 0)
    def _():
        qv_s[...] = jnp.full((n, 128), jnp.inf, jnp.float32)

    pv = None
    pa = None
    for c in range(n // _RC):
        r0 = c * _RC
        x1c = x1_ref[0, r0:r0 + _RC, :]
        dx = x1c[:, 0:1] - x2x
        dy = x1c[:, 1:2] - x2y
        dz = x1c[:, 2:3] - x2z
        d = (dx * dx + dy * dy) + dz * dz

        d3 = d.reshape(_RC // 8, 8, _TM)
        for a in range(_RC // 8):
            da = d3[a]
            ag = c * (_RC // 8) + a
            if pv is None:
                pv = da
                pa = jnp.zeros((8, _TM), jnp.int32)
            else:
                m = da < pv
                pv = jnp.where(m, da, pv)
                pa = jnp.where(m, ag, pa)

        qv = d[:, 0:128]
        qi = lane + mt * _TM
        for t in range(1, _TM // 128):
            dt = d[:, t * 128:(t + 1) * 128]
            m = dt < qv
            qv = jnp.where(m, dt, qv)
            qi = jnp.where(m, lane + (t * 128 + mt * _TM), qi)

        prev = qv_s[r0:r0 + _RC, :]
        m2 = qv < prev
        qv_s[r0:r0 + _RC, :] = jnp.where(m2, qv, prev)
        qi_s[r0:r0 + _RC, :] = jnp.where(m2, qi, qi_s[r0:r0 + _RC, :])

    rowidx = pa * 8 + lax.broadcasted_iota(jnp.int32, (8, _TM), 0)
    fv = jnp.min(pv, axis=0)
    fi = jnp.min(jnp.where(pv == fv[None, :], rowidx, _BIG), axis=0)
    dist2_ref[0, 0, :] = fv
    idx2_ref[0, 0, :] = fi

    @pl.when(mt == nmt - 1)
    def _():
        qvf = qv_s[...]
        qif = qi_s[...]
        gv = jnp.min(qvf, axis=1)
        gi = jnp.min(jnp.where(qvf == gv[:, None], qif, _BIG), axis=1)
        dist1_ref[0, 0, :] = gv
        idx1_ref[0, 0, :] = gi


def kernel(xyz1, xyz2):
    B, N, _ = xyz1.shape
    M = xyz2.shape[1]
    x2t = jnp.transpose(xyz2, (0, 2, 1))
    grid = (B, M // _TM)

    dist1, idx1, dist2, idx2 = pl.pallas_call(
        _chamfer_body,
        grid=grid,
        in_specs=[
            pl.BlockSpec((1, N, 3), lambda b, mt: (b, 0, 0)),
            pl.BlockSpec((1, 3, _TM), lambda b, mt: (b, 0, mt)),
        ],
        out_specs=[
            pl.BlockSpec((1, 1, N), lambda b, mt: (b, 0, 0)),
            pl.BlockSpec((1, 1, N), lambda b, mt: (b, 0, 0)),
            pl.BlockSpec((1, 1, _TM), lambda b, mt: (b, 0, mt)),
            pl.BlockSpec((1, 1, _TM), lambda b, mt: (b, 0, mt)),
        ],
        out_shape=[
            jax.ShapeDtypeStruct((B, 1, N), jnp.float32),
            jax.ShapeDtypeStruct((B, 1, N), jnp.int32),
            jax.ShapeDtypeStruct((B, 1, M), jnp.float32),
            jax.ShapeDtypeStruct((B, 1, M), jnp.int32),
        ],
        scratch_shapes=[
            pltpu.VMEM((N, 128), jnp.float32),
            pltpu.VMEM((N, 128), jnp.int32),
        ],
        compiler_params=pltpu.CompilerParams(
            dimension_semantics=("parallel", "arbitrary"),
        ),
    )(xyz1, x2t)

    return (
        dist1.reshape(B, N),
        dist2.reshape(B, M),
        idx1.reshape(B, N),
        idx2.reshape(B, M),
    )


# speedup vs baseline: 2.5401x; 1.0185x over previous
import jax
import jax.numpy as jnp
from jax import lax
from jax.experimental import pallas as pl
from jax.experimental.pallas import tpu as pltpu

_TM = 512
_RC = 32
_BIG = 2**30


def _chamfer_body(x1_ref, x2t_ref, dist1_ref, idx1_ref, dist2_ref, idx2_ref,
                  qv_s, qi_s):
    mt = pl.program_id(1)
    nmt = pl.num_programs(1)
    n = x1_ref.shape[1]
    x2t = x2t_ref[0]
    x2x = x2t[0:1, :]
    x2y = x2t[1:2, :]
    x2z = x2t[2:3, :]
    lane = lax.broadcasted_iota(jnp.int32, (_RC, 128), 1)
    ibase = [lane + (t * 128 + mt * _TM) for t in range(_TM // 128)]

    @pl.when(mt == 0)
    def _():
        qv_s[...] = jnp.full((n, 128), jnp.inf, jnp.float32)

    pv = None
    pa = None
    for c in range(n // _RC):
        r0 = c * _RC
        x1c = x1_ref[0, r0:r0 + _RC, :]
        dx = x1c[:, 0:1] - x2x
        dy = x1c[:, 1:2] - x2y
        dz = x1c[:, 2:3] - x2z
        d = (dx * dx + dy * dy) + dz * dz

        d3 = d.reshape(_RC // 8, 8, _TM)
        for a in range(_RC // 8):
            da = d3[a]
            ag = c * (_RC // 8) + a
            if pv is None:
                pv = da
                pa = jnp.zeros((8, _TM), jnp.int32)
            else:
                m = da < pv
                pv = jnp.where(m, da, pv)
                pa = jnp.where(m, ag, pa)

        qv = d[:, 0:128]
        qi = ibase[0]
        for t in range(1, _TM // 128):
            dt = d[:, t * 128:(t + 1) * 128]
            m = dt < qv
            qv = jnp.where(m, dt, qv)
            qi = jnp.where(m, ibase[t], qi)

        prev = qv_s[r0:r0 + _RC, :]
        m2 = qv < prev
        qv_s[r0:r0 + _RC, :] = jnp.where(m2, qv, prev)
        qi_s[r0:r0 + _RC, :] = jnp.where(m2, qi, qi_s[r0:r0 + _RC, :])

    rowidx = pa * 8 + lax.broadcasted_iota(jnp.int32, (8, _TM), 0)
    fv = jnp.min(pv, axis=0)
    fi = jnp.min(jnp.where(pv == fv[None, :], rowidx, _BIG), axis=0)
    dist2_ref[0, 0, :] = fv
    idx2_ref[0, 0, :] = fi

    @pl.when(mt == nmt - 1)
    def _():
        qvf = qv_s[...]
        qif = qi_s[...]
        gv = jnp.min(qvf, axis=1, keepdims=True)
        gi = jnp.min(jnp.where(qvf == gv, qif, _BIG), axis=1, keepdims=True)
        dist1_ref[0, :, :] = gv
        idx1_ref[0, :, :] = gi


def kernel(xyz1, xyz2):
    B, N, _ = xyz1.shape
    M = xyz2.shape[1]
    x2t = jnp.transpose(xyz2, (0, 2, 1))
    grid = (B, M // _TM)

    dist1, idx1, dist2, idx2 = pl.pallas_call(
        _chamfer_body,
        grid=grid,
        in_specs=[
            pl.BlockSpec((1, N, 3), lambda b, mt: (b, 0, 0)),
            pl.BlockSpec((1, 3, _TM), lambda b, mt: (b, 0, mt)),
        ],
        out_specs=[
            pl.BlockSpec((1, N, 1), lambda b, mt: (b, 0, 0)),
            pl.BlockSpec((1, N, 1), lambda b, mt: (b, 0, 0)),
            pl.BlockSpec((1, 1, _TM), lambda b, mt: (b, 0, mt)),
            pl.BlockSpec((1, 1, _TM), lambda b, mt: (b, 0, mt)),
        ],
        out_shape=[
            jax.ShapeDtypeStruct((B, N, 1), jnp.float32),
            jax.ShapeDtypeStruct((B, N, 1), jnp.int32),
            jax.ShapeDtypeStruct((B, 1, M), jnp.float32),
            jax.ShapeDtypeStruct((B, 1, M), jnp.int32),
        ],
        scratch_shapes=[
            pltpu.VMEM((N, 128), jnp.float32),
            pltpu.VMEM((N, 128), jnp.int32),
        ],
        compiler_params=pltpu.CompilerParams(
            dimension_semantics=("parallel", "arbitrary"),
        ),
    )(xyz1, x2t)

    return (
        dist1.reshape(B, N),
        dist2.reshape(B, M),
        idx1.reshape(B, N),
        idx2.reshape(B, M),
    )
